# baseline (speedup 1.0000x reference)
import functools

import numpy as np
import jax
import jax.numpy as jnp
from jax import lax
from jax.experimental import pallas as pl
from jax.experimental.pallas import tpu as pltpu
from jax.experimental.pallas import tpu_sc as plsc

D = 768
BATCH = 4
SEQ = 4096
FLAT = BATCH * SEQ
NW = 32
ROWS_PER_W = FLAT // NW
CHUNK = 64
NCHUNK = ROWS_PER_W // CHUNK
LANES = 16
SCALE = float(np.sqrt(np.float32(D)))


def _sin_pe():
    position = np.arange(0, SEQ, dtype=np.float32)[:, None]
    div_term = np.exp(
        np.arange(0, D, 2).astype(np.float32) * (-np.log(10000.0) / D))
    pe = np.zeros((SEQ, D), dtype=np.float32)
    pe[:, 0::2] = np.sin(position * div_term)
    pe[:, 1::2] = np.cos(position * div_term)
    return pe


_PE_NP = _sin_pe()

_MESH = plsc.VectorSubcoreMesh(core_axis_name="c", subcore_axis_name="s")


@functools.partial(
    pl.kernel,
    mesh=_MESH,
    out_type=jax.ShapeDtypeStruct((FLAT, D), jnp.float32),
    scratch_types=[
        pltpu.VMEM((NCHUNK, CHUNK), jnp.int32),
        pltpu.VMEM((CHUNK, D), jnp.float32),
        pltpu.VMEM((CHUNK, D), jnp.float32),
        pltpu.SemaphoreType.DMA,
    ],
)
def _embed(x_hbm, table_hbm, pe_hbm, out_hbm, idx_v, rows_v, pe_v, sem):
    cid = lax.axis_index("c")
    sid = lax.axis_index("s")
    wid = cid * 16 + sid
    base = wid * ROWS_PER_W

    pltpu.sync_copy(x_hbm.at[wid], idx_v)

    def chunk_body(g, carry):
        row0 = base + g * CHUNK
        pos0 = lax.rem(row0, SEQ)
        gather = pltpu.async_copy(table_hbm.at[idx_v.at[g]], rows_v, sem)
        pltpu.sync_copy(pe_hbm.at[pl.ds(pos0, CHUNK)], pe_v)
        gather.wait()

        def row_body(r, c2):
            for j in range(D // LANES):
                sl = (r, pl.ds(j * LANES, LANES))
                rows_v[sl] = rows_v[sl] * SCALE + pe_v[sl]
            return c2

        lax.fori_loop(0, CHUNK, row_body, 0)
        pltpu.sync_copy(rows_v, out_hbm.at[pl.ds(row0, CHUNK)])
        return carry

    lax.fori_loop(0, NCHUNK, chunk_body, 0)


def kernel(x, table):
    xf = x.reshape(NW, NCHUNK, CHUNK).astype(jnp.int32)
    out = _embed(xf, table, jnp.asarray(_PE_NP))
    return out.reshape(BATCH, SEQ, D)
